# Initial kernel scaffold; baseline (speedup 1.0000x reference)
import jax
import jax.numpy as jnp
from jax.experimental import pallas as pl


def kernel(inputs, embedding):
    raise NotImplementedError("write your pallas kernel here")


# speedup vs baseline: 1.5480x; 1.5480x over previous
import functools

import jax
import jax.numpy as jnp
from jax import lax
from jax.experimental import pallas as pl
from jax.experimental.pallas import tpu as pltpu
from jax.experimental.pallas import tpu_sc as plsc

_NC = 2
_NS = 16
_NW = _NC * _NS


@functools.lru_cache(maxsize=None)
def _make_gather(N, D, CH):
    n_per_w = N // _NW
    n_chunks = n_per_w // CH
    assert n_chunks * CH == n_per_w, (N, CH)
    mesh = plsc.VectorSubcoreMesh(core_axis_name="c", subcore_axis_name="s")

    @functools.partial(
        pl.kernel,
        mesh=mesh,
        out_type=jax.ShapeDtypeStruct((N, D), jnp.float32),
        scratch_types=[
            pltpu.VMEM((CH,), jnp.int32),
            pltpu.VMEM((CH, D), jnp.float32),
            pltpu.SemaphoreType.DMA,
        ],
        compiler_params=pltpu.CompilerParams(use_tc_tiling_on_sc=False),
    )
    def k(idx_hbm, tbl_hbm, out_hbm, idx_v, rows_v, sem):
        wid = lax.axis_index("s") * _NC + lax.axis_index("c")
        base = wid * n_per_w

        def chunk(i, carry):
            off = pl.multiple_of(base + i * CH, 8)
            pltpu.sync_copy(idx_hbm.at[pl.ds(off, CH)], idx_v)
            pltpu.async_copy(tbl_hbm.at[idx_v], rows_v, sem).wait()
            pltpu.sync_copy(rows_v, out_hbm.at[pl.ds(off, CH)])
            return carry

        lax.fori_loop(0, n_chunks, chunk, 0)

    return k


def kernel(inputs, embedding):
    B, F = inputs.shape
    V, D = embedding.shape
    N = B * F
    idx = inputs.reshape(N)
    out = _make_gather(N, D, 1024)(idx, embedding)
    return out.reshape(B, F, D)


# speedup vs baseline: 1.5649x; 1.0109x over previous
import functools

import jax
import jax.numpy as jnp
from jax import lax
from jax.experimental import pallas as pl
from jax.experimental.pallas import tpu as pltpu
from jax.experimental.pallas import tpu_sc as plsc

_NC = 2
_NS = 16
_NW = _NC * _NS


@functools.lru_cache(maxsize=None)
def _make_gather(N, D, CH):
    n_per_w = N // _NW
    n_chunks = n_per_w // CH
    assert n_chunks * CH == n_per_w, (N, CH)
    mesh = plsc.VectorSubcoreMesh(core_axis_name="c", subcore_axis_name="s")

    @functools.partial(
        pl.kernel,
        mesh=mesh,
        out_type=jax.ShapeDtypeStruct((N, D), jnp.float32),
        scratch_types=[
            pltpu.VMEM((n_per_w,), jnp.int32),
            pltpu.VMEM((CH, D), jnp.float32),
            pltpu.VMEM((CH, D), jnp.float32),
            pltpu.SemaphoreType.DMA,
            pltpu.SemaphoreType.DMA,
            pltpu.SemaphoreType.DMA,
            pltpu.SemaphoreType.DMA,
        ],
        compiler_params=pltpu.CompilerParams(use_tc_tiling_on_sc=False),
    )
    def k(idx_hbm, tbl_hbm, out_hbm, idx_all, rows0, rows1,
          sem_g0, sem_g1, sem_w0, sem_w1):
        wid = lax.axis_index("s") * _NC + lax.axis_index("c")
        base = pl.multiple_of(wid * n_per_w, 8)
        pltpu.sync_copy(idx_hbm.at[pl.ds(base, n_per_w)], idx_all)

        rows = (rows0, rows1)
        sem_g = (sem_g0, sem_g1)
        sem_w = (sem_w0, sem_w1)

        def start_gather(i):
            return pltpu.async_copy(
                tbl_hbm.at[idx_all.at[pl.ds(i * CH, CH)]],
                rows[i % 2], sem_g[i % 2])

        gathers = [None] * n_chunks
        wbs = [None] * n_chunks
        gathers[0] = start_gather(0)
        for i in range(n_chunks):
            b = i % 2
            gathers[i].wait()
            if i > 0:
                wbs[i - 1].wait()
            if i + 1 < n_chunks:
                gathers[i + 1] = start_gather(i + 1)
            off = pl.multiple_of(base + i * CH, 8)
            wbs[i] = pltpu.async_copy(
                rows[b], out_hbm.at[pl.ds(off, CH)], sem_w[b])
        wbs[n_chunks - 1].wait()

    return k


def kernel(inputs, embedding):
    B, F = inputs.shape
    V, D = embedding.shape
    N = B * F
    idx = inputs.reshape(N)
    out = _make_gather(N, D, 1024)(idx, embedding)
    return out.reshape(B, F, D)


# speedup vs baseline: 1.5750x; 1.0065x over previous
import functools

import jax
import jax.numpy as jnp
from jax import lax
from jax.experimental import pallas as pl
from jax.experimental.pallas import tpu as pltpu
from jax.experimental.pallas import tpu_sc as plsc

_NC = 2
_NS = 16
_NW = _NC * _NS


@functools.lru_cache(maxsize=None)
def _make_gather(N, D, CH, NBUF):
    n_per_w = N // _NW
    n_chunks = n_per_w // CH
    assert n_chunks * CH == n_per_w, (N, CH)
    mesh = plsc.VectorSubcoreMesh(core_axis_name="c", subcore_axis_name="s")

    @functools.partial(
        pl.kernel,
        mesh=mesh,
        out_type=jax.ShapeDtypeStruct((N, D), jnp.float32),
        scratch_types=[
            pltpu.VMEM((n_per_w,), jnp.int32),
            *[pltpu.VMEM((CH, D), jnp.float32) for _ in range(NBUF)],
            *[pltpu.SemaphoreType.DMA for _ in range(2 * NBUF)],
        ],
        compiler_params=pltpu.CompilerParams(use_tc_tiling_on_sc=False),
    )
    def k(idx_hbm, tbl_hbm, out_hbm, idx_all, *bufs_and_sems):
        rows = bufs_and_sems[:NBUF]
        sem_g = bufs_and_sems[NBUF:2 * NBUF]
        sem_w = bufs_and_sems[2 * NBUF:]
        wid = lax.axis_index("s") * _NC + lax.axis_index("c")
        base = pl.multiple_of(wid * n_per_w, 8)
        pltpu.sync_copy(idx_hbm.at[pl.ds(base, n_per_w)], idx_all)

        def start_gather(i):
            return pltpu.async_copy(
                tbl_hbm.at[idx_all.at[pl.ds(i * CH, CH)]],
                rows[i % NBUF], sem_g[i % NBUF])

        gathers = [None] * n_chunks
        wbs = [None] * n_chunks
        for i in range(min(NBUF - 1, n_chunks)):
            gathers[i] = start_gather(i)
        for i in range(n_chunks):
            b = i % NBUF
            gathers[i].wait()
            off = pl.multiple_of(base + i * CH, 8)
            wbs[i] = pltpu.async_copy(
                rows[b], out_hbm.at[pl.ds(off, CH)], sem_w[b])
            nxt = i + NBUF - 1
            if nxt < n_chunks:
                if i >= 1:
                    wbs[i - 1].wait()
                gathers[nxt] = start_gather(nxt)
        for i in range(max(0, n_chunks - NBUF), n_chunks):
            if wbs[i] is not None:
                wbs[i].wait()

    return k


def kernel(inputs, embedding):
    B, F = inputs.shape
    V, D = embedding.shape
    N = B * F
    idx = inputs.reshape(N)
    out = _make_gather(N, D, 832, 4)(idx, embedding)
    return out.reshape(B, F, D)
